# Initial kernel scaffold; baseline (speedup 1.0000x reference)
import jax
import jax.numpy as jnp
from jax.experimental import pallas as pl


def kernel(embed_map, vertices, E_mask):
    raise NotImplementedError("write your pallas kernel here")


# speedup vs baseline: 1.5055x; 1.5055x over previous
import functools

import jax
import jax.numpy as jnp
from jax import lax
from jax.experimental import pallas as pl
from jax.experimental.pallas import tpu as pltpu
from jax.experimental.pallas import tpu_sc as plsc

_VOCAB = 1000000
_D = 32
_B, _H, _W = 4, 512, 512
_HW = _H * _W
_N = _B * _HW

_NC, _NS, _L = 2, 16, 16
_NWORK = _NC * _NS
_NPW = _N // _NWORK
_C = 1024
_CP = _C + 1
_G = 128
_NG = _C // _G
_NCHUNK = _NPW // _C


def _body(emb_hbm, vert_hbm, mask_hbm, out_hbm,
          idx_v, mask_v, rows_v, obuf_v, gsem, osem):
    wid = lax.axis_index("s") * _NC + lax.axis_index("c")
    gbase = wid * _NPW
    b = gbase // _HW
    hw0 = gbase - b * _HW

    iota = lax.iota(jnp.int32, _L)

    def chunk_body(g, _):
        ebase = gbase + g * _C

        pltpu.sync_copy(
            vert_hbm.at[pl.ds(pl.multiple_of(ebase // _G, 8), _NG)], idx_v)
        pltpu.sync_copy(mask_hbm.at[pl.ds(ebase, _C)], mask_v)

        cps = []
        for j in range(_NG):
            cps.append(pltpu.async_copy(
                emb_hbm.at[idx_v.at[j]],
                rows_v.at[pl.ds(j * _G, _G)],
                gsem))
        for cp in cps:
            cp.wait()

        def tr_body(i, _):
            c0 = i * _L
            mv = mask_v[pl.ds(c0, _L)]
            for ci in range(_L):
                c = c0 + ci
                csplat = jnp.full((_L,), c, dtype=jnp.int32)
                m = mv[ci]
                r0 = rows_v[c, pl.ds(0, _L)]
                r1 = rows_v[c, pl.ds(_L, _L)]
                plsc.store_scatter(obuf_v, [iota, csplat], r0 * m)
                plsc.store_scatter(obuf_v, [iota + _L, csplat], r1 * m)
            return ()

        lax.fori_loop(0, _C // _L, tr_body, (), unroll=False)

        hw = hw0 + g * _C
        ocs = []
        for d in range(_D):
            ocs.append(pltpu.async_copy(
                obuf_v.at[d, pl.ds(0, _C)],
                out_hbm.at[b, d, pl.ds(hw, _C)],
                osem))
        for cp in ocs:
            cp.wait()
        return ()

    lax.fori_loop(0, _NCHUNK, chunk_body, (), unroll=False)


@jax.jit
def _run(embed_map, vert2d, maskflat):
    mesh = plsc.VectorSubcoreMesh(core_axis_name="c", subcore_axis_name="s")
    f = pl.kernel(
        _body,
        out_type=jax.ShapeDtypeStruct((_B, _D, _HW), jnp.float32),
        mesh=mesh,
        scratch_types=[
            pltpu.VMEM((_NG, _G), jnp.int32),
            pltpu.VMEM((_C,), jnp.float32),
            pltpu.VMEM((_C, _D), jnp.float32),
            pltpu.VMEM((_D, _CP), jnp.float32),
            pltpu.SemaphoreType.DMA,
            pltpu.SemaphoreType.DMA,
        ],
        compiler_params=pltpu.CompilerParams(
            needs_layout_passes=False, use_tc_tiling_on_sc=False),
    )
    return f(embed_map, vert2d, maskflat)


def kernel(embed_map, vertices, E_mask):
    vert2d = vertices.reshape(_N // _G, _G)
    maskflat = E_mask.reshape(_N)
    out = _run(embed_map, vert2d, maskflat)
    return out.reshape(_B, _D, _H, _W)
